# baseline (speedup 1.0000x reference)
import functools

import jax
import jax.numpy as jnp
from jax import lax
from jax.experimental import pallas as pl
from jax.experimental.pallas import tpu as pltpu
from jax.experimental.pallas import tpu_sc as plsc

N = 10000
E = 320000
G = 128
H = 128
NLAYERS = 5
NFEAT = 9
VOCAB = 64
EPS = 1e-5

NC = 2
NS = 16
NW = NC * NS

CH = 128
EPT = -(-E // NW)
NCHUNK = 80
EALL = NW * NCHUNK * CH
SW = 8
NWIN = NCHUNK // SW
ACC_ROWS = 10240
ROWS_PT = ACC_ROWS // NS

NPAD = 10240
NODES_PT = NPAD // NW
ATOM_CH = 64
ATOM_NCH = NODES_PT // ATOM_CH

_mesh = plsc.VectorSubcoreMesh(core_axis_name="c", subcore_axis_name="s")


def _fill_2d(ref, rows, cols, value):
    def body(r, carry):
        for cb in range(cols // 16):
            ref[r, pl.ds(cb * 16, 16)] = jnp.full((16,), value, jnp.float32)
        return carry
    lax.fori_loop(0, rows, body, 0)


@functools.partial(
    pl.kernel,
    out_type=(
        jax.ShapeDtypeStruct((NPAD, H), jnp.float32),
        jax.ShapeDtypeStruct((NC, ACC_ROWS, H), jnp.float32),
    ),
    mesh=_mesh,
    scratch_types=[
        pltpu.VMEM((NFEAT, ATOM_NCH, ATOM_CH), jnp.int32),
        pltpu.VMEM((ATOM_CH, H), jnp.float32),
        pltpu.VMEM((NCHUNK, CH), jnp.int32),
        pltpu.VMEM((CH, H), jnp.float32),
        pltpu.VMEM_SHARED((ACC_ROWS, H), jnp.float32),
        pltpu.SemaphoreType.DMA,
    ],
)
def _sc_prepass(table_hbm, xidx_hbm, dst_hbm, h0_hbm, deg_hbm,
                idx_v, abuf, dst_v, obuf, deg_sp, sem):
    c = lax.axis_index("c")
    s = lax.axis_index("s")
    w = s * NC + c

    _fill_2d(obuf, CH, H, 0.0)
    for k in range(ROWS_PT // CH):
        pltpu.sync_copy(obuf, deg_sp.at[pl.ds(s * ROWS_PT + k * CH, CH)])

    pltpu.sync_copy(xidx_hbm.at[w], idx_v)
    for k in range(ATOM_NCH):
        for f in range(NFEAT):
            pltpu.async_copy(table_hbm.at[idx_v.at[f, k]], abuf, sem,
                             add=(f > 0)).wait()
        pltpu.sync_copy(abuf,
                        h0_hbm.at[pl.ds(w * NODES_PT + k * ATOM_CH, ATOM_CH)])

    _fill_2d(obuf, CH, H, 1.0)
    pltpu.sync_copy(dst_hbm.at[c, s], dst_v)
    plsc.subcore_barrier()

    def deg_body(j, carry):
        pltpu.sync_copy(obuf, deg_sp.at[dst_v.at[j]], add=True)
        return carry
    lax.fori_loop(0, NCHUNK, deg_body, 0)

    plsc.subcore_barrier()
    for k in range(ROWS_PT // CH):
        pltpu.sync_copy(deg_sp.at[pl.ds(s * ROWS_PT + k * CH, CH)], obuf)
        pltpu.sync_copy(obuf, deg_hbm.at[c, pl.ds(s * ROWS_PT + k * CH, CH)])


@functools.partial(
    pl.kernel,
    out_type=jax.ShapeDtypeStruct((NC, ACC_ROWS, H), jnp.float32),
    mesh=_mesh,
    scratch_types=[
        pltpu.VMEM((NCHUNK, CH), jnp.int32),
        pltpu.VMEM((SW, CH), jnp.int32),
        pltpu.VMEM((SW, CH), jnp.int32),
        pltpu.VMEM((CH, H), jnp.float32),
        pltpu.VMEM((CH, H), jnp.float32),
        pltpu.VMEM_SHARED((ACC_ROWS, H), jnp.float32),
        pltpu.SemaphoreType.DMA,
        pltpu.SemaphoreType.DMA,
        pltpu.SemaphoreType.DMA,
        pltpu.SemaphoreType.DMA,
    ],
)
def _sc_scatter(hw_hbm, src_hbm, dst_hbm, acc_hbm,
                dst_v, sw0, sw1, gb0, gb1, acc_sp,
                gs0, gs1, ws0, ws1):
    c = lax.axis_index("c")
    s = lax.axis_index("s")
    gsems = (gs0, gs1)
    gbufs = (gb0, gb1)
    sws = (sw0, sw1)
    wsems = (ws0, ws1)

    _fill_2d(gb0, CH, H, 0.0)
    for k in range(ROWS_PT // CH):
        pltpu.sync_copy(gb0, acc_sp.at[pl.ds(s * ROWS_PT + k * CH, CH)])
    pltpu.sync_copy(dst_hbm.at[c, s], dst_v)
    plsc.subcore_barrier()

    def win_copy(w):
        return pltpu.make_async_copy(
            src_hbm.at[c, s, pl.ds(w * SW, SW)], sws[w % 2], wsems[w % 2])

    def gather(j, w):
        return pltpu.make_async_copy(
            hw_hbm.at[sws[w % 2].at[j % SW]], gbufs[j % 2], gsems[j % 2])

    win_copy(0).start()
    win_copy(1).start()
    win_copy(0).wait()
    gather(0, 0).start()
    gather(1, 0).start()
    for j in range(NCHUNK):
        w = j // SW
        gather(j, w).wait()
        pltpu.sync_copy(gbufs[j % 2], acc_sp.at[dst_v.at[j]], add=True)
        if j % SW == 0 and 2 <= (w + 1) < NWIN:
            win_copy(w + 1).start()
        jn = j + 2
        if jn < NCHUNK:
            wn = jn // SW
            if jn % SW == 0:
                win_copy(wn).wait()
            gather(jn, wn).start()

    plsc.subcore_barrier()
    for k in range(ROWS_PT // CH):
        pltpu.sync_copy(acc_sp.at[pl.ds(s * ROWS_PT + k * CH, CH)], gb0)
        pltpu.sync_copy(gb0, acc_hbm.at[c, pl.ds(s * ROWS_PT + k * CH, CH)])


def _tc_stage0_body(h0_ref, deg_ref, w_ref, hw_ref, dinv_ref):
    degsum = deg_ref[0, :N, 0:1] + deg_ref[1, :N, 0:1] + 1.0
    dinv = lax.rsqrt(degsum)
    hw = jnp.dot(h0_ref[:N, :], w_ref[...], preferred_element_type=jnp.float32,
                 precision=lax.Precision.HIGHEST)
    hw_ref[...] = hw * dinv
    dinv_ref[...] = dinv


def _tc_stage0(h0, deg, w0):
    return pl.pallas_call(
        _tc_stage0_body,
        out_shape=(
            jax.ShapeDtypeStruct((N, H), jnp.float32),
            jax.ShapeDtypeStruct((N, 1), jnp.float32),
        ),
    )(h0, deg, w0)


def _tc_mid_body(acc_ref, hwp_ref, dinv_ref, b_ref, sc_ref, of_ref, w_ref,
                 out_ref):
    dinv = dinv_ref[...]
    a = acc_ref[0, :N, :] + acc_ref[1, :N, :] + hwp_ref[...]
    h = dinv * a + b_ref[...]
    mean = jnp.mean(h, axis=0, keepdims=True)
    d = h - mean
    var = jnp.mean(d * d, axis=0, keepdims=True)
    h = d * lax.rsqrt(var + EPS) * sc_ref[...] + of_ref[...]
    h = jnp.maximum(h, 0.0)
    hw = jnp.dot(h, w_ref[...], preferred_element_type=jnp.float32,
                 precision=lax.Precision.HIGHEST)
    out_ref[...] = hw * dinv


def _tc_mid(acc, hwp, dinv, b, scale, offset, w):
    return pl.pallas_call(
        _tc_mid_body,
        out_shape=jax.ShapeDtypeStruct((N, H), jnp.float32),
    )(acc, hwp, dinv, b, scale, offset, w)


def _tc_final_body(acc_ref, hwp_ref, dinv_ref, b_ref, batch_ref, lw_ref,
                   lb_ref, out_ref):
    a = acc_ref[0, :N, :] + acc_ref[1, :N, :] + hwp_ref[...]
    h = dinv_ref[...] * a + b_ref[...]
    gids = lax.broadcasted_iota(jnp.int32, (1, G), 1)
    sel = (batch_ref[...] == gids).astype(jnp.float32)
    tdims = (((0,), (0,)), ((), ()))
    sums = lax.dot_general(sel, h, dimension_numbers=tdims,
                           preferred_element_type=jnp.float32,
                           precision=lax.Precision.HIGHEST)
    cnt = lax.dot_general(sel, jnp.ones((N, 1), jnp.float32),
                          dimension_numbers=tdims,
                          preferred_element_type=jnp.float32,
                          precision=lax.Precision.HIGHEST)
    pooled = sums / jnp.maximum(cnt, 1.0)
    out_ref[...] = (jnp.dot(pooled, lw_ref[...],
                            preferred_element_type=jnp.float32,
                            precision=lax.Precision.HIGHEST) + lb_ref[...])


def _tc_final(acc, hwp, dinv, b, batch_col, lw, lb):
    return pl.pallas_call(
        _tc_final_body,
        out_shape=jax.ShapeDtypeStruct((G, 1), jnp.float32),
    )(acc, hwp, dinv, b, batch_col, lw, lb)


def kernel(x, edge_index, batch, atom_emb, conv_W, conv_b, bn_scale,
           bn_offset, lin_W, lin_b):
    i32 = jnp.int32
    f32 = jnp.float32

    src = edge_index[0].astype(i32)
    dst = edge_index[1].astype(i32)
    pad_e = EALL - E
    pad_dst = N + (jnp.arange(pad_e, dtype=i32) % (ACC_ROWS - N))
    srcp = jnp.concatenate([src, jnp.zeros((pad_e,), i32)])
    dstp = jnp.concatenate([dst, pad_dst])
    srcp = srcp.reshape(NC, NS, NCHUNK, CH)
    dstp = dstp.reshape(NC, NS, NCHUNK, CH)

    xpad = jnp.concatenate(
        [x.astype(i32), jnp.zeros((NPAD - N, NFEAT), i32)], axis=0)
    xidx = xpad + (jnp.arange(NFEAT, dtype=i32) * VOCAB)[None, :]
    xidx = xidx.reshape(NW, ATOM_NCH, ATOM_CH, NFEAT).transpose(0, 3, 1, 2)
    table = atom_emb.astype(f32).reshape(NFEAT * VOCAB, H)

    h0, deg = _sc_prepass(table, xidx, dstp)
    hwp, dinv = _tc_stage0(h0, deg, conv_W[0].astype(f32))

    for i in range(1, NLAYERS):
        acc = _sc_scatter(hwp, srcp, dstp)
        hwp = _tc_mid(acc, hwp, dinv,
                      conv_b[i - 1].astype(f32).reshape(1, H),
                      bn_scale[i - 1].astype(f32).reshape(1, H),
                      bn_offset[i - 1].astype(f32).reshape(1, H),
                      conv_W[i].astype(f32))

    acc = _sc_scatter(hwp, srcp, dstp)
    out = _tc_final(acc, hwp, dinv,
                    conv_b[NLAYERS - 1].astype(f32).reshape(1, H),
                    batch.astype(i32).reshape(N, 1),
                    lin_W.astype(f32), lin_b.astype(f32).reshape(1, 1))
    return out


# speedup vs baseline: 1.2572x; 1.2572x over previous
import functools

import jax
import jax.numpy as jnp
from jax import lax
from jax.experimental import pallas as pl
from jax.experimental.pallas import tpu as pltpu
from jax.experimental.pallas import tpu_sc as plsc

N = 10000
E = 320000
G = 128
H = 128
NLAYERS = 5
NFEAT = 9
VOCAB = 64
EPS = 1e-5

NC = 2
NS = 16
NW = NC * NS

CH = 128
EPT = -(-E // NW)
NCHUNK = -(-EPT // CH)
EALL = NW * NCHUNK * CH
ACC_ROWS = 10240
ROWS_PT = ACC_ROWS // NS

NPAD = 10240
NODES_PT = NPAD // NW
ATOM_CH = 64
ATOM_NCH = NODES_PT // ATOM_CH

_mesh = plsc.VectorSubcoreMesh(core_axis_name="c", subcore_axis_name="s")


def _fill_2d(ref, rows, cols, value):
    def body(r, carry):
        for cb in range(cols // 16):
            ref[r, pl.ds(cb * 16, 16)] = jnp.full((16,), value, jnp.float32)
        return carry
    lax.fori_loop(0, rows, body, 0)


@functools.partial(
    pl.kernel,
    out_type=(
        jax.ShapeDtypeStruct((NPAD, H), jnp.float32),
        jax.ShapeDtypeStruct((NC, ACC_ROWS, 16), jnp.float32),
    ),
    mesh=_mesh,
    scratch_types=[
        pltpu.VMEM((NFEAT, ATOM_NCH, ATOM_CH), jnp.int32),
        pltpu.VMEM((ATOM_CH, H), jnp.float32),
        pltpu.VMEM((NCHUNK, CH), jnp.int32),
        pltpu.VMEM((CH, 16), jnp.float32),
        pltpu.VMEM_SHARED((ACC_ROWS, 16), jnp.float32),
        pltpu.SemaphoreType.DMA,
    ],
)
def _sc_prepass(table_hbm, xidx_hbm, dst_hbm, h0_hbm, deg_hbm,
                idx_v, abuf, dst_v, w16, deg_sp, sem):
    c = lax.axis_index("c")
    s = lax.axis_index("s")
    w = s * NC + c

    _fill_2d(w16, CH, 16, 0.0)
    for k in range(ROWS_PT // CH):
        pltpu.sync_copy(w16, deg_sp.at[pl.ds(s * ROWS_PT + k * CH, CH)])

    pltpu.sync_copy(xidx_hbm.at[w], idx_v)
    for k in range(ATOM_NCH):
        for f in range(NFEAT):
            pltpu.async_copy(table_hbm.at[idx_v.at[f, k]], abuf, sem,
                             add=(f > 0)).wait()
        pltpu.sync_copy(abuf,
                        h0_hbm.at[pl.ds(w * NODES_PT + k * ATOM_CH, ATOM_CH)])

    _fill_2d(w16, CH, 16, 1.0)
    pltpu.sync_copy(dst_hbm.at[c, s], dst_v)
    plsc.subcore_barrier()

    def deg_body(j, carry):
        pltpu.sync_copy(w16, deg_sp.at[dst_v.at[j]], add=True)
        return carry
    lax.fori_loop(0, NCHUNK, deg_body, 0)

    plsc.subcore_barrier()
    for k in range(ROWS_PT // CH):
        pltpu.sync_copy(deg_sp.at[pl.ds(s * ROWS_PT + k * CH, CH)], w16)
        pltpu.sync_copy(w16, deg_hbm.at[c, pl.ds(s * ROWS_PT + k * CH, CH)])


@functools.partial(
    pl.kernel,
    out_type=jax.ShapeDtypeStruct((NC, ACC_ROWS, H), jnp.float32),
    mesh=_mesh,
    scratch_types=[
        pltpu.VMEM((NCHUNK, CH), jnp.int32),
        pltpu.VMEM((NCHUNK, CH), jnp.int32),
        pltpu.VMEM((CH, H), jnp.float32),
        pltpu.VMEM_SHARED((ACC_ROWS, H), jnp.float32),
        pltpu.SemaphoreType.DMA,
    ],
)
def _sc_scatter(hw_hbm, src_hbm, dst_hbm, acc_hbm,
                src_v, dst_v, gbuf, acc_sp, sem):
    c = lax.axis_index("c")
    s = lax.axis_index("s")

    _fill_2d(gbuf, CH, H, 0.0)
    for k in range(ROWS_PT // CH):
        pltpu.sync_copy(gbuf, acc_sp.at[pl.ds(s * ROWS_PT + k * CH, CH)])
    pltpu.sync_copy(src_hbm.at[c, s], src_v)
    pltpu.sync_copy(dst_hbm.at[c, s], dst_v)
    plsc.subcore_barrier()

    def body(j, carry):
        pltpu.async_copy(hw_hbm.at[src_v.at[j]], gbuf, sem).wait()
        pltpu.sync_copy(gbuf, acc_sp.at[dst_v.at[j]], add=True)
        return carry
    lax.fori_loop(0, NCHUNK, body, 0)

    plsc.subcore_barrier()
    for k in range(ROWS_PT // CH):
        pltpu.sync_copy(acc_sp.at[pl.ds(s * ROWS_PT + k * CH, CH)], gbuf)
        pltpu.sync_copy(gbuf, acc_hbm.at[c, pl.ds(s * ROWS_PT + k * CH, CH)])


def _tc_stage0_body(h0_ref, deg_ref, w_ref, hw_ref, dinv_ref):
    degsum = deg_ref[0, :N, 0:1] + deg_ref[1, :N, 0:1] + 1.0
    dinv = lax.rsqrt(degsum)
    hw = jnp.dot(h0_ref[:N, :], w_ref[...], preferred_element_type=jnp.float32,
                 precision=lax.Precision.HIGHEST)
    hw_ref[...] = hw * dinv
    dinv_ref[...] = dinv


def _tc_stage0(h0, deg, w0):
    return pl.pallas_call(
        _tc_stage0_body,
        out_shape=(
            jax.ShapeDtypeStruct((N, H), jnp.float32),
            jax.ShapeDtypeStruct((N, 1), jnp.float32),
        ),
    )(h0, deg, w0)


def _tc_mid_body(acc_ref, hwp_ref, dinv_ref, b_ref, sc_ref, of_ref, w_ref,
                 out_ref):
    dinv = dinv_ref[...]
    a = acc_ref[0, :N, :] + acc_ref[1, :N, :] + hwp_ref[...]
    h = dinv * a + b_ref[...]
    mean = jnp.mean(h, axis=0, keepdims=True)
    d = h - mean
    var = jnp.mean(d * d, axis=0, keepdims=True)
    h = d * lax.rsqrt(var + EPS) * sc_ref[...] + of_ref[...]
    h = jnp.maximum(h, 0.0)
    hw = jnp.dot(h, w_ref[...], preferred_element_type=jnp.float32,
                 precision=lax.Precision.HIGHEST)
    out_ref[...] = hw * dinv


def _tc_mid(acc, hwp, dinv, b, scale, offset, w):
    return pl.pallas_call(
        _tc_mid_body,
        out_shape=jax.ShapeDtypeStruct((N, H), jnp.float32),
    )(acc, hwp, dinv, b, scale, offset, w)


def _tc_final_body(acc_ref, hwp_ref, dinv_ref, b_ref, batch_ref, lw_ref,
                   lb_ref, out_ref):
    a = acc_ref[0, :N, :] + acc_ref[1, :N, :] + hwp_ref[...]
    h = dinv_ref[...] * a + b_ref[...]
    gids = lax.broadcasted_iota(jnp.int32, (1, G), 1)
    sel = (batch_ref[...] == gids).astype(jnp.float32)
    tdims = (((0,), (0,)), ((), ()))
    sums = lax.dot_general(sel, h, dimension_numbers=tdims,
                           preferred_element_type=jnp.float32,
                           precision=lax.Precision.HIGHEST)
    cnt = lax.dot_general(sel, jnp.ones((N, 1), jnp.float32),
                          dimension_numbers=tdims,
                          preferred_element_type=jnp.float32,
                          precision=lax.Precision.HIGHEST)
    pooled = sums / jnp.maximum(cnt, 1.0)
    out_ref[...] = (jnp.dot(pooled, lw_ref[...],
                            preferred_element_type=jnp.float32,
                            precision=lax.Precision.HIGHEST) + lb_ref[...])


def _tc_final(acc, hwp, dinv, b, batch_col, lw, lb):
    return pl.pallas_call(
        _tc_final_body,
        out_shape=jax.ShapeDtypeStruct((G, 1), jnp.float32),
    )(acc, hwp, dinv, b, batch_col, lw, lb)


def kernel(x, edge_index, batch, atom_emb, conv_W, conv_b, bn_scale,
           bn_offset, lin_W, lin_b):
    i32 = jnp.int32
    f32 = jnp.float32

    src = edge_index[0].astype(i32)
    dst = edge_index[1].astype(i32)
    pad_e = EALL - E
    pad_dst = N + (jnp.arange(pad_e, dtype=i32) % (ACC_ROWS - N))
    srcp = jnp.concatenate([src, jnp.zeros((pad_e,), i32)])
    dstp = jnp.concatenate([dst, pad_dst])
    srcp = srcp.reshape(NC, NS, NCHUNK, CH)
    dstp = dstp.reshape(NC, NS, NCHUNK, CH)

    xpad = jnp.concatenate(
        [x.astype(i32), jnp.zeros((NPAD - N, NFEAT), i32)], axis=0)
    xidx = xpad + (jnp.arange(NFEAT, dtype=i32) * VOCAB)[None, :]
    xidx = xidx.reshape(NW, ATOM_NCH, ATOM_CH, NFEAT).transpose(0, 3, 1, 2)
    table = atom_emb.astype(f32).reshape(NFEAT * VOCAB, H)

    h0, deg = _sc_prepass(table, xidx, dstp)
    hwp, dinv = _tc_stage0(h0, deg, conv_W[0].astype(f32))

    for i in range(1, NLAYERS):
        acc = _sc_scatter(hwp, srcp, dstp)
        hwp = _tc_mid(acc, hwp, dinv,
                      conv_b[i - 1].astype(f32).reshape(1, H),
                      bn_scale[i - 1].astype(f32).reshape(1, H),
                      bn_offset[i - 1].astype(f32).reshape(1, H),
                      conv_W[i].astype(f32))

    acc = _sc_scatter(hwp, srcp, dstp)
    out = _tc_final(acc, hwp, dinv,
                    conv_b[NLAYERS - 1].astype(f32).reshape(1, H),
                    batch.astype(i32).reshape(N, 1),
                    lin_W.astype(f32), lin_b.astype(f32).reshape(1, 1))
    return out


# speedup vs baseline: 1.3755x; 1.0942x over previous
import functools

import jax
import jax.numpy as jnp
from jax import lax
from jax.experimental import pallas as pl
from jax.experimental.pallas import tpu as pltpu
from jax.experimental.pallas import tpu_sc as plsc

N = 10000
E = 320000
G = 128
H = 128
NLAYERS = 5
NFEAT = 9
VOCAB = 64
EPS = 1e-5

NC = 2
NS = 16
NW = NC * NS

CH = 128
EPT = -(-E // NW)
NCHUNK = -(-EPT // CH)
EALL = NW * NCHUNK * CH
ACC_ROWS = 10240
ROWS_PT = ACC_ROWS // NS

NPAD = 10240
NODES_PT = NPAD // NW
ATOM_CH = 64
ATOM_NCH = NODES_PT // ATOM_CH

_mesh = plsc.VectorSubcoreMesh(core_axis_name="c", subcore_axis_name="s")


def _fill_2d(ref, rows, cols, value):
    def body(r, carry):
        for cb in range(cols // 16):
            ref[r, pl.ds(cb * 16, 16)] = jnp.full((16,), value, jnp.float32)
        return carry
    lax.fori_loop(0, rows, body, 0)


@functools.partial(
    pl.kernel,
    out_type=(
        jax.ShapeDtypeStruct((NPAD, H), jnp.float32),
        jax.ShapeDtypeStruct((NC, ACC_ROWS, H), jnp.float32),
    ),
    mesh=_mesh,
    scratch_types=[
        pltpu.VMEM((NFEAT, ATOM_NCH, ATOM_CH), jnp.int32),
        pltpu.VMEM((ATOM_CH, H), jnp.float32),
        pltpu.VMEM((NCHUNK, CH), jnp.int32),
        pltpu.VMEM((CH, H), jnp.float32),
        pltpu.VMEM_SHARED((ACC_ROWS, H), jnp.float32),
        pltpu.SemaphoreType.DMA,
    ],
)
def _sc_prepass(table_hbm, xidx_hbm, dst_hbm, h0_hbm, deg_hbm,
                idx_v, abuf, dst_v, obuf, deg_sp, sem):
    c = lax.axis_index("c")
    s = lax.axis_index("s")
    w = s * NC + c

    _fill_2d(obuf, CH, H, 0.0)
    for k in range(ROWS_PT // CH):
        pltpu.sync_copy(obuf, deg_sp.at[pl.ds(s * ROWS_PT + k * CH, CH)])

    pltpu.sync_copy(xidx_hbm.at[w], idx_v)
    for k in range(ATOM_NCH):
        for f in range(NFEAT):
            pltpu.async_copy(table_hbm.at[idx_v.at[f, k]], abuf, sem,
                             add=(f > 0)).wait()
        pltpu.sync_copy(abuf,
                        h0_hbm.at[pl.ds(w * NODES_PT + k * ATOM_CH, ATOM_CH)])

    _fill_2d(obuf, CH, H, 1.0)
    pltpu.sync_copy(dst_hbm.at[c, s], dst_v)
    plsc.subcore_barrier()

    def deg_body(j, carry):
        pltpu.sync_copy(obuf, deg_sp.at[dst_v.at[j]], add=True)
        return carry
    lax.fori_loop(0, NCHUNK, deg_body, 0)

    plsc.subcore_barrier()
    for k in range(ROWS_PT // CH):
        pltpu.sync_copy(deg_sp.at[pl.ds(s * ROWS_PT + k * CH, CH)], obuf)
        pltpu.sync_copy(obuf, deg_hbm.at[c, pl.ds(s * ROWS_PT + k * CH, CH)])


@functools.partial(
    pl.kernel,
    out_type=jax.ShapeDtypeStruct((NC, ACC_ROWS, H), jnp.float32),
    mesh=_mesh,
    scratch_types=[
        pltpu.VMEM((NCHUNK, CH), jnp.int32),
        pltpu.VMEM((NCHUNK, CH), jnp.int32),
        pltpu.VMEM((CH, H), jnp.float32),
        pltpu.VMEM_SHARED((ACC_ROWS, H), jnp.float32),
        pltpu.SemaphoreType.DMA,
    ],
)
def _sc_scatter(hw_hbm, src_hbm, dst_hbm, acc_hbm,
                src_v, dst_v, gbuf, acc_sp, sem):
    c = lax.axis_index("c")
    s = lax.axis_index("s")

    _fill_2d(gbuf, CH, H, 0.0)
    for k in range(ROWS_PT // CH):
        pltpu.sync_copy(gbuf, acc_sp.at[pl.ds(s * ROWS_PT + k * CH, CH)])
    pltpu.sync_copy(src_hbm.at[c, s], src_v)
    pltpu.sync_copy(dst_hbm.at[c, s], dst_v)
    plsc.subcore_barrier()

    def body(j, carry):
        pltpu.async_copy(hw_hbm.at[src_v.at[j]], gbuf, sem).wait()
        pltpu.sync_copy(gbuf, acc_sp.at[dst_v.at[j]], add=True)
        return carry
    lax.fori_loop(0, NCHUNK, body, 0)

    plsc.subcore_barrier()
    for k in range(ROWS_PT // CH):
        pltpu.sync_copy(acc_sp.at[pl.ds(s * ROWS_PT + k * CH, CH)], gbuf)
        pltpu.sync_copy(gbuf, acc_hbm.at[c, pl.ds(s * ROWS_PT + k * CH, CH)])


def _rsqrt(x):
    y = lax.rsqrt(x)
    return y * (1.5 - 0.5 * x * y * y)


def _tc_stage0_body(h0_ref, deg_ref, w_ref, hw_ref, dinv_ref):
    degsum = deg_ref[0, :N, 0:1] + deg_ref[1, :N, 0:1] + 1.0
    dinv = _rsqrt(degsum)
    hw = jnp.dot(h0_ref[:N, :], w_ref[...], preferred_element_type=jnp.float32)
    hw_ref[...] = hw * dinv
    dinv_ref[...] = dinv


def _tc_stage0(h0, deg, w0):
    return pl.pallas_call(
        _tc_stage0_body,
        out_shape=(
            jax.ShapeDtypeStruct((N, H), jnp.float32),
            jax.ShapeDtypeStruct((N, 1), jnp.float32),
        ),
    )(h0, deg, w0)


def _tc_mid_body(acc_ref, hwp_ref, dinv_ref, b_ref, sc_ref, of_ref, w_ref,
                 out_ref):
    dinv = dinv_ref[...]
    a = acc_ref[0, :N, :] + acc_ref[1, :N, :] + hwp_ref[...]
    h = dinv * a + b_ref[...]
    mean = jnp.mean(h, axis=0, keepdims=True)
    d = h - mean
    var = jnp.mean(d * d, axis=0, keepdims=True)
    h = d * _rsqrt(var + EPS) * sc_ref[...] + of_ref[...]
    h = jnp.maximum(h, 0.0)
    hw = jnp.dot(h, w_ref[...], preferred_element_type=jnp.float32)
    out_ref[...] = hw * dinv


def _tc_mid(acc, hwp, dinv, b, scale, offset, w):
    return pl.pallas_call(
        _tc_mid_body,
        out_shape=jax.ShapeDtypeStruct((N, H), jnp.float32),
    )(acc, hwp, dinv, b, scale, offset, w)


def _tc_final_body(acc_ref, hwp_ref, dinv_ref, b_ref, batch_ref, lw_ref,
                   lb_ref, out_ref):
    a = acc_ref[0, :N, :] + acc_ref[1, :N, :] + hwp_ref[...]
    h = dinv_ref[...] * a + b_ref[...]
    gids = lax.broadcasted_iota(jnp.int32, (1, G), 1)
    sel = (batch_ref[...] == gids).astype(jnp.float32)
    tdims = (((0,), (0,)), ((), ()))
    sums = lax.dot_general(sel, h, dimension_numbers=tdims,
                           preferred_element_type=jnp.float32,
                           precision=lax.Precision.HIGHEST)
    cnt = lax.dot_general(sel, jnp.ones((N, 1), jnp.float32),
                          dimension_numbers=tdims,
                          preferred_element_type=jnp.float32,
                          precision=lax.Precision.HIGHEST)
    pooled = sums / jnp.maximum(cnt, 1.0)
    out_ref[...] = (jnp.dot(pooled, lw_ref[...],
                            preferred_element_type=jnp.float32) + lb_ref[...])


def _tc_final(acc, hwp, dinv, b, batch_col, lw, lb):
    return pl.pallas_call(
        _tc_final_body,
        out_shape=jax.ShapeDtypeStruct((G, 1), jnp.float32),
    )(acc, hwp, dinv, b, batch_col, lw, lb)


def kernel(x, edge_index, batch, atom_emb, conv_W, conv_b, bn_scale,
           bn_offset, lin_W, lin_b):
    i32 = jnp.int32
    f32 = jnp.float32

    src = edge_index[0].astype(i32)
    dst = edge_index[1].astype(i32)
    pad_e = EALL - E
    pad_dst = N + (jnp.arange(pad_e, dtype=i32) % (ACC_ROWS - N))
    srcp = jnp.concatenate([src, jnp.zeros((pad_e,), i32)])
    dstp = jnp.concatenate([dst, pad_dst])
    order = jnp.argsort(dstp)
    srcp = srcp[order].reshape(CH, EALL // CH).T
    dstp = dstp[order].reshape(CH, EALL // CH).T
    srcp = srcp.reshape(NC, NS, NCHUNK, CH)
    dstp = dstp.reshape(NC, NS, NCHUNK, CH)

    xpad = jnp.concatenate(
        [x.astype(i32), jnp.zeros((NPAD - N, NFEAT), i32)], axis=0)
    xidx = xpad + (jnp.arange(NFEAT, dtype=i32) * VOCAB)[None, :]
    xidx = xidx.reshape(NW, ATOM_NCH, ATOM_CH, NFEAT).transpose(0, 3, 1, 2)
    table = atom_emb.astype(f32).reshape(NFEAT * VOCAB, H)

    h0, deg = _sc_prepass(table, xidx, dstp)
    hwp, dinv = _tc_stage0(h0, deg, conv_W[0].astype(f32))

    for i in range(1, NLAYERS):
        acc = _sc_scatter(hwp, srcp, dstp)
        hwp = _tc_mid(acc, hwp, dinv,
                      conv_b[i - 1].astype(f32).reshape(1, H),
                      bn_scale[i - 1].astype(f32).reshape(1, H),
                      bn_offset[i - 1].astype(f32).reshape(1, H),
                      conv_W[i].astype(f32))

    acc = _sc_scatter(hwp, srcp, dstp)
    out = _tc_final(acc, hwp, dinv,
                    conv_b[NLAYERS - 1].astype(f32).reshape(1, H),
                    batch.astype(i32).reshape(N, 1),
                    lin_W.astype(f32), lin_b.astype(f32).reshape(1, 1))
    return out
